# baseline (speedup 1.0000x reference)
import functools

import jax
import jax.numpy as jnp
from jax import lax
from jax.experimental import pallas as pl
from jax.experimental.pallas import tpu as pltpu
from jax.experimental.pallas import tpu_sc as plsc

_VOCAB = 100
_H = 128

_NC = 2
_NS = 16
_NW = _NC * _NS

_CHUNK = 128
_CPW = 25
_NBUF = 6
_NPAD = _NW * _CPW * _CHUNK
_TOTAL_CHUNKS = _NW * _CPW
_N = 100000
_LAST_FULL = (_N - (_NW - 1) * _CPW * _CHUNK) // _CHUNK
_LAST_TAIL = _N - (_NW - 1) * _CPW * _CHUNK - _LAST_FULL * _CHUNK


def _proj_body(emb_ref, w_ref, b_ref, out_ref):
    out_ref[:] = (
        lax.dot_general(
            emb_ref[:],
            w_ref[:],
            (((1,), (1,)), ((), ())),
            preferred_element_type=jnp.float32,
            precision=lax.Precision.HIGHEST,
        )
        + b_ref[:]
    )


def _project_table(embed_table, W, b):
    return pl.pallas_call(
        _proj_body,
        out_shape=jax.ShapeDtypeStruct((_VOCAB, _H), jnp.float32),
    )(embed_table, W, b.reshape(1, _H))


_mesh = plsc.VectorSubcoreMesh(core_axis_name="c", subcore_axis_name="s")


@functools.partial(
    pl.kernel,
    mesh=_mesh,
    out_type=jax.ShapeDtypeStruct((_N, _H), jnp.float32),
    scratch_types=[
        pltpu.VMEM_SHARED((_VOCAB, _H), jnp.float32),
        pltpu.VMEM((_CPW, _CHUNK), jnp.int32),
        pltpu.VMEM((_NBUF, _CHUNK, _H), jnp.float32),
        pltpu.SemaphoreType.DMA,
        pltpu.SemaphoreType.DMA,
        pltpu.SemaphoreType.DMA,
        pltpu.SemaphoreType.DMA,
        pltpu.SemaphoreType.DMA,
        pltpu.SemaphoreType.DMA,
        pltpu.SemaphoreType.DMA,
        pltpu.SemaphoreType.DMA,
        pltpu.SemaphoreType.DMA,
        pltpu.SemaphoreType.DMA,
        pltpu.SemaphoreType.DMA,
        pltpu.SemaphoreType.DMA,
    ],
)
def _gather_kernel(
    proj_hbm,
    idx_hbm,
    out_hbm,
    proj_v,
    idx_v,
    rows_v,
    g0, g1, g2, g3, g4, g5,
    w0, w1, w2, w3, w4, w5,
):
    gsem = (g0, g1, g2, g3, g4, g5)
    wsem = (w0, w1, w2, w3, w4, w5)
    wid = lax.axis_index("s") * _NC + lax.axis_index("c")
    base = wid * _CPW
    @pl.when(lax.axis_index("s") == 0)
    def _():
        pltpu.sync_copy(proj_hbm, proj_v)

    plsc.subcore_barrier()
    my_proj = proj_v
    pltpu.sync_copy(idx_hbm.at[wid], idx_v)

    def start_gather(c, j):
        pltpu.async_copy(my_proj.at[idx_v.at[c]], rows_v.at[j], gsem[j])

    def wait_gather(c, j):
        pltpu.make_async_copy(my_proj.at[idx_v.at[c]], rows_v.at[j], gsem[j]).wait()

    def run_chunks(nfull, tail_rows):
        total = nfull + (1 if tail_rows else 0)

        def write_refs(c, j, rows):
            src = rows_v.at[j] if rows == _CHUNK else rows_v.at[j].at[pl.ds(0, rows)]
            dst = out_hbm.at[pl.ds((base + c) * _CHUNK, rows)]
            return src, dst

        def start_write(c, j, rows=_CHUNK):
            src, dst = write_refs(c, j, rows)
            pltpu.async_copy(src, dst, wsem[j])

        def wait_write(c, j, rows=_CHUNK):
            src, dst = write_refs(c, j, rows)
            pltpu.make_async_copy(src, dst, wsem[j]).wait()

        for j in range(min(_NBUF - 1, total)):
            start_gather(j, j)

        def group(g, carry):
            for j in range(_NBUF):
                c = g * _NBUF + j
                jn = (j + _NBUF - 1) % _NBUF
                wait_gather(c, j)
                start_write(c, j)
                n = c + _NBUF - 1

                @pl.when(n < total)
                def _():
                    @pl.when(c > 0)
                    def _():
                        wait_write(c - 1, jn)

                    start_gather(n, jn)

            return carry

        ngroups = total // _NBUF
        if total % _NBUF == 0 and tail_rows:
            ngroups -= 1
        lax.fori_loop(0, ngroups, group, 0)

        for c in range(ngroups * _NBUF, total):
            rows = tail_rows if (tail_rows and c == total - 1) else _CHUNK
            wait_gather(c, c % _NBUF)
            start_write(c, c % _NBUF, rows)

        for c in range(max(0, total - _NBUF), total):
            rows = tail_rows if (tail_rows and c == total - 1) else _CHUNK
            wait_write(c, c % _NBUF, rows)

    @pl.when(wid < _NW - 1)
    def _():
        run_chunks(_CPW, 0)

    @pl.when(wid == _NW - 1)
    def _():
        run_chunks(_LAST_FULL, _LAST_TAIL)


def kernel(atom_types, embed_table, W, b):
    n = atom_types.shape[0]
    proj = _project_table(embed_table, W, b)
    idx = jnp.pad(atom_types.astype(jnp.int32), (0, _NPAD - n))
    return _gather_kernel(proj, idx.reshape(_NW, _CPW, _CHUNK))


# speedup vs baseline: 1.0266x; 1.0266x over previous
import functools

import jax
import jax.numpy as jnp
from jax import lax
from jax.experimental import pallas as pl
from jax.experimental.pallas import tpu as pltpu
from jax.experimental.pallas import tpu_sc as plsc

_VOCAB = 100
_H = 128
_N = 100000

_NC = 2
_NS = 16
_NW = _NC * _NS

_CHUNK = 128
_NBUF = 4
_NFULL = 24
_TAIL_A = 56
_TAIL_B = 48
_SPLIT = 20
_ROWS_A = _NFULL * _CHUNK + _TAIL_A
_ROWS_B = _NFULL * _CHUNK + _TAIL_B
assert _SPLIT * _ROWS_A + (_NW - _SPLIT) * _ROWS_B == _N


def _proj_body(emb_ref, w_ref, b_ref, out_ref):
    out_ref[:] = (
        lax.dot_general(
            emb_ref[:],
            w_ref[:],
            (((1,), (1,)), ((), ())),
            preferred_element_type=jnp.float32,
            precision=lax.Precision.HIGHEST,
        )
        + b_ref[:]
    )


def _project_table(embed_table, W, b):
    return pl.pallas_call(
        _proj_body,
        out_shape=jax.ShapeDtypeStruct((_VOCAB, _H), jnp.float32),
    )(embed_table, W, b.reshape(1, _H))


_mesh = plsc.VectorSubcoreMesh(core_axis_name="c", subcore_axis_name="s")


@functools.partial(
    pl.kernel,
    mesh=_mesh,
    out_type=jax.ShapeDtypeStruct((_N, _H), jnp.float32),
    scratch_types=[
        pltpu.VMEM_SHARED((_VOCAB, _H), jnp.float32),
        pltpu.VMEM((_ROWS_A,), jnp.int32),
        pltpu.VMEM((_NBUF, _CHUNK, _H), jnp.float32),
        pltpu.SemaphoreType.DMA,
        pltpu.SemaphoreType.DMA,
        pltpu.SemaphoreType.DMA,
        pltpu.SemaphoreType.DMA,
        pltpu.SemaphoreType.DMA,
        pltpu.SemaphoreType.DMA,
        pltpu.SemaphoreType.DMA,
        pltpu.SemaphoreType.DMA,
    ],
)
def _gather_kernel(
    proj_hbm, idx_hbm, out_hbm, proj_v, idx_v, rows_v, g0, g1, g2, g3, w0, w1, w2, w3
):
    gsem = (g0, g1, g2, g3)
    wsem = (w0, w1, w2, w3)
    wid = lax.axis_index("s") * _NC + lax.axis_index("c")

    @pl.when(lax.axis_index("s") == 0)
    def _():
        pltpu.sync_copy(proj_hbm, proj_v)

    plsc.subcore_barrier()

    def run_worker(base_row, nrows, tail_rows):
        pltpu.sync_copy(
            idx_hbm.at[pl.ds(base_row, nrows)], idx_v.at[pl.ds(0, nrows)]
        )
        total = _NFULL + 1

        def gather_refs(c, j, rows):
            idx = idx_v.at[pl.ds(c * _CHUNK, rows)]
            dst = rows_v.at[j] if rows == _CHUNK else rows_v.at[j].at[pl.ds(0, rows)]
            return proj_v.at[idx], dst

        def start_gather(c, j, rows=_CHUNK):
            src, dst = gather_refs(c, j, rows)
            pltpu.async_copy(src, dst, gsem[j])

        def wait_gather(c, j, rows=_CHUNK):
            src, dst = gather_refs(c, j, rows)
            pltpu.make_async_copy(src, dst, gsem[j]).wait()

        def write_refs(c, j, rows):
            src = rows_v.at[j] if rows == _CHUNK else rows_v.at[j].at[pl.ds(0, rows)]
            dst = out_hbm.at[pl.ds(base_row + c * _CHUNK, rows)]
            return src, dst

        def start_write(c, j, rows=_CHUNK):
            src, dst = write_refs(c, j, rows)
            pltpu.async_copy(src, dst, wsem[j])

        def wait_write(c, j, rows=_CHUNK):
            src, dst = write_refs(c, j, rows)
            pltpu.make_async_copy(src, dst, wsem[j]).wait()

        for j in range(_NBUF - 1):
            start_gather(j, j)

        def group(g, carry):
            for j in range(_NBUF):
                c = g * _NBUF + j
                jn = (j + _NBUF - 1) % _NBUF
                wait_gather(c, j)
                start_write(c, j)
                n = c + _NBUF - 1

                @pl.when(n < _NFULL)
                def _():
                    @pl.when(c > 0)
                    def _():
                        wait_write(c - 1, jn)

                    start_gather(n, jn)

            return carry

        ngroups = _NFULL // _NBUF
        lax.fori_loop(0, ngroups, group, 0)

        c = total - 1
        j = c % _NBUF
        wait_write(c - _NBUF, j)
        start_gather(c, j, tail_rows)
        wait_gather(c, j, tail_rows)
        start_write(c, j, tail_rows)

        for cc in range(total - _NBUF, total):
            rows = tail_rows if cc == total - 1 else _CHUNK
            wait_write(cc, cc % _NBUF, rows)

    @pl.when(wid < _SPLIT)
    def _():
        run_worker(wid * _ROWS_A, _ROWS_A, _TAIL_A)

    @pl.when(wid >= _SPLIT)
    def _():
        run_worker(
            _SPLIT * _ROWS_A + (wid - _SPLIT) * _ROWS_B, _ROWS_B, _TAIL_B
        )


def kernel(atom_types, embed_table, W, b):
    proj = _project_table(embed_table, W, b)
    return _gather_kernel(proj, atom_types.astype(jnp.int32))


# speedup vs baseline: 1.0451x; 1.0180x over previous
import functools

import jax
import jax.numpy as jnp
from jax import lax
from jax.experimental import pallas as pl
from jax.experimental.pallas import tpu as pltpu
from jax.experimental.pallas import tpu_sc as plsc

_VOCAB = 100
_H = 128
_N = 100000

_NC = 2
_NS = 16
_NW = _NC * _NS

_CHUNK = 128
_NBUF = 4
_NFULL = 24
_TAIL_A = 56
_TAIL_B = 48
_SPLIT = 20
_ROWS_A = _NFULL * _CHUNK + _TAIL_A
_ROWS_B = _NFULL * _CHUNK + _TAIL_B
assert _SPLIT * _ROWS_A + (_NW - _SPLIT) * _ROWS_B == _N


def _proj_body(emb_ref, w_ref, b_ref, out_ref):
    out_ref[:] = (
        lax.dot_general(
            emb_ref[:],
            w_ref[:],
            (((1,), (1,)), ((), ())),
            preferred_element_type=jnp.float32,
            precision=lax.Precision.HIGHEST,
        )
        + b_ref[:]
    )


def _project_table(embed_table, W, b):
    return pl.pallas_call(
        _proj_body,
        out_shape=jax.ShapeDtypeStruct((_VOCAB, _H), jnp.float32),
    )(embed_table, W, b.reshape(1, _H))


_mesh = plsc.VectorSubcoreMesh(core_axis_name="c", subcore_axis_name="s")


@functools.partial(
    pl.kernel,
    mesh=_mesh,
    out_type=jax.ShapeDtypeStruct((_N, _H), jnp.float32),
    scratch_types=[
        pltpu.VMEM_SHARED((_VOCAB, _H), jnp.float32),
        pltpu.VMEM((_ROWS_A,), jnp.int32),
        pltpu.VMEM((_NBUF, _CHUNK, _H), jnp.float32),
        pltpu.SemaphoreType.DMA,
        pltpu.SemaphoreType.DMA,
        pltpu.SemaphoreType.DMA,
        pltpu.SemaphoreType.DMA,
        pltpu.SemaphoreType.DMA,
        pltpu.SemaphoreType.DMA,
        pltpu.SemaphoreType.DMA,
        pltpu.SemaphoreType.DMA,
        pltpu.SemaphoreType.DMA,
    ],
)
def _gather_kernel(
    proj_hbm,
    idx_hbm,
    out_hbm,
    proj_v,
    idx_v,
    rows_v,
    psem,
    g0, g1, g2, g3,
    w0, w1, w2, w3,
):
    gsem = (g0, g1, g2, g3)
    wsem = (w0, w1, w2, w3)
    sid = lax.axis_index("s")
    wid = sid * _NC + lax.axis_index("c")

    @pl.when(sid == 0)
    def _():
        pltpu.async_copy(proj_hbm, proj_v, psem)

    def run_worker(base_row, nrows, tail_rows):
        pltpu.sync_copy(
            idx_hbm.at[pl.ds(base_row, nrows)], idx_v.at[pl.ds(0, nrows)]
        )

        @pl.when(sid == 0)
        def _():
            pltpu.make_async_copy(proj_hbm, proj_v, psem).wait()

        plsc.subcore_barrier()
        total = _NFULL + 1

        def gather_refs(c, j, rows):
            idx = idx_v.at[pl.ds(c * _CHUNK, rows)]
            dst = rows_v.at[j] if rows == _CHUNK else rows_v.at[j].at[pl.ds(0, rows)]
            return proj_v.at[idx], dst

        def start_gather(c, j, rows=_CHUNK):
            src, dst = gather_refs(c, j, rows)
            pltpu.async_copy(src, dst, gsem[j])

        def wait_gather(c, j, rows=_CHUNK):
            src, dst = gather_refs(c, j, rows)
            pltpu.make_async_copy(src, dst, gsem[j]).wait()

        def write_refs(c, j, rows):
            src = rows_v.at[j] if rows == _CHUNK else rows_v.at[j].at[pl.ds(0, rows)]
            dst = out_hbm.at[pl.ds(base_row + c * _CHUNK, rows)]
            return src, dst

        def start_write(c, j, rows=_CHUNK):
            src, dst = write_refs(c, j, rows)
            pltpu.async_copy(src, dst, wsem[j])

        def wait_write(c, j, rows=_CHUNK):
            src, dst = write_refs(c, j, rows)
            pltpu.make_async_copy(src, dst, wsem[j]).wait()

        for j in range(_NBUF - 1):
            start_gather(j, j)

        def group(g, carry):
            for j in range(_NBUF):
                c = g * _NBUF + j
                jn = (j + _NBUF - 1) % _NBUF
                wait_gather(c, j)
                start_write(c, j)
                n = c + _NBUF - 1

                @pl.when(n < _NFULL)
                def _():
                    @pl.when(c > 0)
                    def _():
                        wait_write(c - 1, jn)

                    start_gather(n, jn)

            return carry

        ngroups = _NFULL // _NBUF
        lax.fori_loop(0, ngroups, group, 0)

        c = total - 1
        j = c % _NBUF
        wait_write(c - _NBUF, j)
        start_gather(c, j, tail_rows)
        wait_gather(c, j, tail_rows)
        start_write(c, j, tail_rows)

        for cc in range(total - _NBUF, total):
            rows = tail_rows if cc == total - 1 else _CHUNK
            wait_write(cc, cc % _NBUF, rows)

    @pl.when(wid < _SPLIT)
    def _():
        run_worker(wid * _ROWS_A, _ROWS_A, _TAIL_A)

    @pl.when(wid >= _SPLIT)
    def _():
        run_worker(
            _SPLIT * _ROWS_A + (wid - _SPLIT) * _ROWS_B, _ROWS_B, _TAIL_B
        )


def kernel(atom_types, embed_table, W, b):
    proj = _project_table(embed_table, W, b)
    return _gather_kernel(proj, atom_types.astype(jnp.int32))
